# baseline (speedup 1.0000x reference)
import functools

import jax
import jax.numpy as jnp
from jax.experimental import pallas as pl
from jax.experimental.pallas import tpu as pltpu

N_CLS = 7
BH = 512
TR = 8
U = 4
NPAR = 2
LOG2E = 1.4426950408889634
LN2 = 0.6931471805599453


def _fold(v):
    return (v[:, 0:128] + v[:, 128:256]) + (v[:, 256:384] + v[:, 384:512])


def _body(x_ref, t_ref, o_ref, acci_ref, accf_ref, *, nbh, wdt):
    b = pl.program_id(0)
    p = pl.program_id(1)

    @pl.when(jnp.logical_and(b == 0, p == 0))
    def _init():
        acci_ref[...] = jnp.zeros_like(acci_ref)
        accf_ref[...] = jnp.zeros_like(accf_ref)

    def tile(i, _):
        for j in range(U):
            r = pl.multiple_of((i * U + j) * TR, TR)
            a = j % NPAR
            xs = [x_ref[0, c, pl.ds(r, TR), :] for c in range(N_CLS)]
            t = t_ref[0, pl.ds(r, TR), :]

            m = xs[0]
            for c in range(1, N_CLS):
                m = jnp.maximum(m, xs[c])

            es = [jnp.exp2(xs[c] * LOG2E) for c in range(N_CLS)]
            s = ((es[0] + es[1]) + (es[2] + es[3])) + ((es[4] + es[5]) + es[6])
            lse = LN2 * jnp.log2(s)

            oh = [t == c for c in range(N_CLS - 1)]
            xt = xs[N_CLS - 1]
            for c in range(N_CLS - 1):
                xt = jnp.where(oh[c], xs[c], xt)
            nll = lse - xt
            packed = jnp.where(xt < m, 4097, 1)

            for c in range(N_CLS - 1):
                acci_ref[a, c] += _fold(jnp.where(oh[c], packed, 0))
                accf_ref[a, c] += _fold(jnp.where(oh[c], nll, 0.0))
            acci_ref[a, N_CLS - 1] += _fold(packed)
            accf_ref[a, N_CLS - 1] += _fold(nll)
        return 0

    jax.lax.fori_loop(0, x_ref.shape[2] // (TR * U), tile, 0)

    @pl.when(jnp.logical_and(b == pl.num_programs(0) - 1, p == nbh - 1))
    def _final():
        cs = 0.0
        fs = 0.0
        ns = 0.0
        num = 0.0
        den = 0.0
        cnt = [None] * N_CLS
        fn = [None] * N_CLS
        nl = [None] * N_CLS
        for c in range(N_CLS - 1):
            pk = acci_ref[0, c] + acci_ref[1, c]
            cnt[c] = jnp.sum(pk & 4095).astype(jnp.float32)
            fn[c] = jnp.sum(pk >> 12).astype(jnp.float32)
            nl[c] = jnp.sum(accf_ref[0, c] + accf_ref[1, c])
            cs = cs + cnt[c]
            fs = fs + fn[c]
            ns = ns + nl[c]
        tot = acci_ref[0, N_CLS - 1] + acci_ref[1, N_CLS - 1]
        cnt[N_CLS - 1] = jnp.sum(tot & 4095).astype(jnp.float32) - cs
        fn[N_CLS - 1] = jnp.sum(tot >> 12).astype(jnp.float32) - fs
        nl[N_CLS - 1] = jnp.sum(accf_ref[0, N_CLS - 1] + accf_ref[1, N_CLS - 1]) - ns
        for c in range(N_CLS):
            gt = jnp.where(cnt[c] > 0.0, cnt[c], 1.0)
            fnc = jnp.where(fn[c] > 0.0, fn[c], 1.0)
            w = fnc / gt
            num = num + w * nl[c]
            den = den + w * cnt[c]
        o_ref[0, 0] = num / den


@jax.jit
def kernel(input, target):
    bsz, ncls, h, wdt = input.shape
    nbh = h // BH

    out = pl.pallas_call(
        functools.partial(_body, nbh=nbh, wdt=wdt),
        grid=(bsz, nbh),
        in_specs=[
            pl.BlockSpec((1, ncls, BH, wdt), lambda b, p: (b, 0, p, 0)),
            pl.BlockSpec((1, BH, wdt), lambda b, p: (b, p, 0)),
        ],
        out_specs=pl.BlockSpec(memory_space=pltpu.SMEM),
        out_shape=jax.ShapeDtypeStruct((1, 1), jnp.float32),
        scratch_shapes=[
            pltpu.VMEM((NPAR, N_CLS, TR, 128), jnp.int32),
            pltpu.VMEM((NPAR, N_CLS, TR, 128), jnp.float32),
        ],
        compiler_params=pltpu.CompilerParams(
            dimension_semantics=("arbitrary", "arbitrary"),
        ),
    )(input, target)
    return out[0, 0]
